# baseline (device time: 80317 ns/iter reference)
import functools

import jax
import jax.numpy as jnp
from jax import lax
from jax.experimental import pallas as pl
from jax.experimental.pallas import tpu as pltpu

T = 1024
D = 1024
V_PER = 8192
NQ = 4
Q = T // NQ


def _ring_to_xz(t):
    tx = t // 2
    tz = (tx + t) % 2
    return tx, tz


def kernel(ids, E):
    my_y = lax.axis_index("y")
    local = ids - my_y * V_PER
    mask = (local >= 0) & (local < V_PER)
    safe = jnp.where(mask, local, 0)
    partial = jnp.where(mask[:, None], E[safe, :], jnp.float32(0.0))

    def body(p_ref, o_ref, other_ref, send_sems, recv_sems):
        my_x = lax.axis_index("x")
        my_y = lax.axis_index("y")
        my_z = lax.axis_index("z")

        r = 2 * my_x + (my_x + my_z) % 2
        rt = (r + 1) % NQ
        lt = (r + 3) % NQ

        rx, rz = _ring_to_xz(rt)
        lx, lz = _ring_to_xz(lt)
        y_peer = (my_x, 1 - my_y, my_z)
        right = (rx, my_y, rz)
        left = (lx, my_y, lz)

        barrier = pltpu.get_barrier_semaphore()
        for dev in (y_peer, right, left):
            pl.semaphore_signal(
                barrier, inc=1, device_id=dev,
                device_id_type=pl.DeviceIdType.MESH,
            )
        pl.semaphore_wait(barrier, 3)

        rdma_y = pltpu.make_async_remote_copy(
            src_ref=p_ref.at[pl.ds(r * Q, Q)],
            dst_ref=other_ref.at[r],
            send_sem=send_sems.at[0],
            recv_sem=recv_sems.at[0],
            device_id=y_peer,
            device_id_type=pl.DeviceIdType.MESH,
        )
        rdma_y.start()
        rdma_y.wait()

        rdma_r1 = pltpu.make_async_remote_copy(
            src_ref=other_ref.at[r],
            dst_ref=other_ref.at[r],
            send_sem=send_sems.at[1],
            recv_sem=recv_sems.at[1],
            device_id=right,
            device_id_type=pl.DeviceIdType.MESH,
        )
        rdma_l1 = pltpu.make_async_remote_copy(
            src_ref=other_ref.at[r],
            dst_ref=other_ref.at[r],
            send_sem=send_sems.at[2],
            recv_sem=recv_sems.at[2],
            device_id=left,
            device_id_type=pl.DeviceIdType.MESH,
        )
        rdma_r1.start()
        rdma_l1.start()
        rdma_r1.wait()
        rdma_l1.wait()

        rdma_r2 = pltpu.make_async_remote_copy(
            src_ref=other_ref.at[lt],
            dst_ref=other_ref.at[lt],
            send_sem=send_sems.at[3],
            recv_sem=recv_sems.at[3],
            device_id=right,
            device_id_type=pl.DeviceIdType.MESH,
        )
        rdma_r2.start()
        rdma_r2.wait()

        for q in range(NQ):
            o_ref[q * Q:(q + 1) * Q, :] = (
                p_ref[q * Q:(q + 1) * Q, :] + other_ref[q]
            )

        @functools.partial(
            pl.run_scoped, exit_sem=pltpu.SemaphoreType.REGULAR
        )
        def _(exit_sem):
            for dev in (y_peer, right, left):
                pl.semaphore_signal(
                    exit_sem, inc=1, device_id=dev,
                    device_id_type=pl.DeviceIdType.MESH,
                )
            pl.semaphore_wait(exit_sem, 3)

    return pl.pallas_call(
        body,
        out_shape=jax.ShapeDtypeStruct((T, D), jnp.float32),
        in_specs=[pl.BlockSpec(memory_space=pltpu.VMEM)],
        out_specs=pl.BlockSpec(memory_space=pltpu.VMEM),
        scratch_shapes=[
            pltpu.VMEM((NQ, Q, D), jnp.float32),
            pltpu.SemaphoreType.DMA((4,)),
            pltpu.SemaphoreType.DMA((4,)),
        ],
        compiler_params=pltpu.CompilerParams(collective_id=0),
    )(partial)


# device time: 54500 ns/iter; 1.4737x vs baseline; 1.4737x over previous
import functools

import jax
import jax.numpy as jnp
from jax import lax
from jax.experimental import pallas as pl
from jax.experimental.pallas import tpu as pltpu

T = 1024
D = 1024
V_PER = 8192
NQ = 4
Q = T // NQ


def _ring_to_xz(t):
    tx = t // 2
    tz = (tx + t) % 2
    return tx, tz


def kernel(ids, E):
    my_x = lax.axis_index("x")
    my_y = lax.axis_index("y")
    my_z = lax.axis_index("z")
    r = 2 * my_x + (my_x + my_z) % 2

    my_ids = lax.dynamic_slice(ids, (r * Q,), (Q,))
    local = my_ids - my_y * V_PER
    mask = (local >= 0) & (local < V_PER)
    safe = jnp.where(mask, local, 0)
    pq = jnp.where(mask[:, None], E[safe, :], jnp.float32(0.0))

    def body(pq_ref, o_ref, rbuf, qbuf, send_sems, recv_sems):
        my_x = lax.axis_index("x")
        my_y = lax.axis_index("y")
        my_z = lax.axis_index("z")

        r = 2 * my_x + (my_x + my_z) % 2
        rt = (r + 1) % NQ
        lt = (r + 3) % NQ

        rx, rz = _ring_to_xz(rt)
        lx, lz = _ring_to_xz(lt)
        y_peer = (my_x, 1 - my_y, my_z)
        right = (rx, my_y, rz)
        left = (lx, my_y, lz)

        barrier = pltpu.get_barrier_semaphore()
        for dev in (y_peer, right, left):
            pl.semaphore_signal(
                barrier, inc=1, device_id=dev,
                device_id_type=pl.DeviceIdType.MESH,
            )
        pl.semaphore_wait(barrier, 3)

        rdma_y = pltpu.make_async_remote_copy(
            src_ref=pq_ref,
            dst_ref=rbuf,
            send_sem=send_sems.at[0],
            recv_sem=recv_sems.at[0],
            device_id=y_peer,
            device_id_type=pl.DeviceIdType.MESH,
        )
        rdma_y.start()
        rdma_y.wait()

        qbuf[r, :, :] = pq_ref[:, :] + rbuf[:, :]

        rdma_r1 = pltpu.make_async_remote_copy(
            src_ref=qbuf.at[r],
            dst_ref=qbuf.at[r],
            send_sem=send_sems.at[1],
            recv_sem=recv_sems.at[1],
            device_id=right,
            device_id_type=pl.DeviceIdType.MESH,
        )
        rdma_l1 = pltpu.make_async_remote_copy(
            src_ref=qbuf.at[r],
            dst_ref=qbuf.at[r],
            send_sem=send_sems.at[2],
            recv_sem=recv_sems.at[2],
            device_id=left,
            device_id_type=pl.DeviceIdType.MESH,
        )
        rdma_r1.start()
        rdma_l1.start()
        rdma_r1.wait()
        rdma_l1.wait()

        rdma_r2 = pltpu.make_async_remote_copy(
            src_ref=qbuf.at[lt],
            dst_ref=qbuf.at[lt],
            send_sem=send_sems.at[3],
            recv_sem=recv_sems.at[3],
            device_id=right,
            device_id_type=pl.DeviceIdType.MESH,
        )
        rdma_r2.start()
        rdma_r2.wait()

        for q in range(NQ):
            o_ref[q * Q:(q + 1) * Q, :] = qbuf[q, :, :]

        @functools.partial(
            pl.run_scoped, exit_sem=pltpu.SemaphoreType.REGULAR
        )
        def _(exit_sem):
            for dev in (y_peer, right, left):
                pl.semaphore_signal(
                    exit_sem, inc=1, device_id=dev,
                    device_id_type=pl.DeviceIdType.MESH,
                )
            pl.semaphore_wait(exit_sem, 3)

    return pl.pallas_call(
        body,
        out_shape=jax.ShapeDtypeStruct((T, D), jnp.float32),
        in_specs=[pl.BlockSpec(memory_space=pltpu.VMEM)],
        out_specs=pl.BlockSpec(memory_space=pltpu.VMEM),
        scratch_shapes=[
            pltpu.VMEM((Q, D), jnp.float32),
            pltpu.VMEM((NQ, Q, D), jnp.float32),
            pltpu.SemaphoreType.DMA((4,)),
            pltpu.SemaphoreType.DMA((4,)),
        ],
        compiler_params=pltpu.CompilerParams(collective_id=0),
    )(pq)


# device time: 38312 ns/iter; 2.0964x vs baseline; 1.4225x over previous
import functools

import jax
import jax.numpy as jnp
from jax import lax
from jax.experimental import pallas as pl
from jax.experimental.pallas import tpu as pltpu

T = 1024
D = 1024
V_PER = 8192
NQ = 4
Q = T // NQ
C = 4
CR = Q // C
H = C // 2


def _ring_to_xz(t):
    tx = t // 2
    tz = (tx + t) % 2
    return tx, tz


def kernel(ids, E):
    my_x = lax.axis_index("x")
    my_y = lax.axis_index("y")
    my_z = lax.axis_index("z")
    r = 2 * my_x + (my_x + my_z) % 2

    my_ids = lax.dynamic_slice(ids, (r * Q,), (Q,))
    local = my_ids - my_y * V_PER
    mask = (local >= 0) & (local < V_PER)
    safe = jnp.where(mask, local, 0)
    pq = jnp.where(mask[:, None], E[safe, :], jnp.float32(0.0))

    def body(pq_ref, o_ref, rbuf, qbuf,
             ysend, yrecv, s1rs, s1rr, s1ls, s1lr, s2s, s2r):
        my_x = lax.axis_index("x")
        my_y = lax.axis_index("y")
        my_z = lax.axis_index("z")

        r = 2 * my_x + (my_x + my_z) % 2
        rt = (r + 1) % NQ
        lt = (r + 3) % NQ

        rx, rz = _ring_to_xz(rt)
        lx, lz = _ring_to_xz(lt)
        y_peer = (my_x, 1 - my_y, my_z)
        right = (rx, my_y, rz)
        left = (lx, my_y, lz)

        barrier = pltpu.get_barrier_semaphore()
        for dev in (y_peer, right, left):
            pl.semaphore_signal(
                barrier, inc=1, device_id=dev,
                device_id_type=pl.DeviceIdType.MESH,
            )
        pl.semaphore_wait(barrier, 3)

        y_rdmas = []
        for c in range(C):
            sl = pl.ds(c * CR, CR)
            rd = pltpu.make_async_remote_copy(
                src_ref=pq_ref.at[sl],
                dst_ref=rbuf.at[sl],
                send_sem=ysend.at[c],
                recv_sem=yrecv.at[c],
                device_id=y_peer,
                device_id_type=pl.DeviceIdType.MESH,
            )
            rd.start()
            y_rdmas.append(rd)

        s1r_rdmas = []
        s1l_rdmas = []
        for c in range(C):
            sl = pl.ds(c * CR, CR)
            y_rdmas[c].wait_recv()
            qbuf[r, c * CR:(c + 1) * CR, :] = (
                pq_ref[c * CR:(c + 1) * CR, :] + rbuf[c * CR:(c + 1) * CR, :]
            )
            rdr = pltpu.make_async_remote_copy(
                src_ref=qbuf.at[r, sl],
                dst_ref=qbuf.at[r, sl],
                send_sem=s1rs.at[c],
                recv_sem=s1rr.at[c],
                device_id=right,
                device_id_type=pl.DeviceIdType.MESH,
            )
            rdl = pltpu.make_async_remote_copy(
                src_ref=qbuf.at[r, sl],
                dst_ref=qbuf.at[r, sl],
                send_sem=s1ls.at[c],
                recv_sem=s1lr.at[c],
                device_id=left,
                device_id_type=pl.DeviceIdType.MESH,
            )
            rdr.start()
            rdl.start()
            s1r_rdmas.append(rdr)
            s1l_rdmas.append(rdl)

        s2_rdmas = []
        for c in range(C):
            sl = pl.ds(c * CR, CR)
            if c < H:
                s1r_rdmas[c].wait_recv()
                rd = pltpu.make_async_remote_copy(
                    src_ref=qbuf.at[lt, sl],
                    dst_ref=qbuf.at[lt, sl],
                    send_sem=s2s.at[c],
                    recv_sem=s2r.at[c],
                    device_id=right,
                    device_id_type=pl.DeviceIdType.MESH,
                )
            else:
                s1l_rdmas[c].wait_recv()
                rd = pltpu.make_async_remote_copy(
                    src_ref=qbuf.at[rt, sl],
                    dst_ref=qbuf.at[rt, sl],
                    send_sem=s2s.at[c],
                    recv_sem=s2r.at[c],
                    device_id=left,
                    device_id_type=pl.DeviceIdType.MESH,
                )
            rd.start()
            s2_rdmas.append(rd)

        for c in range(C):
            if c < H:
                s1l_rdmas[c].wait_recv()
            else:
                s1r_rdmas[c].wait_recv()
            s2_rdmas[c].wait_recv()
            y_rdmas[c].wait_send()
            s1r_rdmas[c].wait_send()
            s1l_rdmas[c].wait_send()
            s2_rdmas[c].wait_send()

        for q in range(NQ):
            o_ref[q * Q:(q + 1) * Q, :] = qbuf[q, :, :]

        @functools.partial(
            pl.run_scoped, exit_sem=pltpu.SemaphoreType.REGULAR
        )
        def _(exit_sem):
            for dev in (y_peer, right, left):
                pl.semaphore_signal(
                    exit_sem, inc=1, device_id=dev,
                    device_id_type=pl.DeviceIdType.MESH,
                )
            pl.semaphore_wait(exit_sem, 3)

    return pl.pallas_call(
        body,
        out_shape=jax.ShapeDtypeStruct((T, D), jnp.float32),
        in_specs=[pl.BlockSpec(memory_space=pltpu.VMEM)],
        out_specs=pl.BlockSpec(memory_space=pltpu.VMEM),
        scratch_shapes=[
            pltpu.VMEM((Q, D), jnp.float32),
            pltpu.VMEM((NQ, Q, D), jnp.float32),
            pltpu.SemaphoreType.DMA((C,)),
            pltpu.SemaphoreType.DMA((C,)),
            pltpu.SemaphoreType.DMA((C,)),
            pltpu.SemaphoreType.DMA((C,)),
            pltpu.SemaphoreType.DMA((C,)),
            pltpu.SemaphoreType.DMA((C,)),
            pltpu.SemaphoreType.DMA((C,)),
            pltpu.SemaphoreType.DMA((C,)),
        ],
        compiler_params=pltpu.CompilerParams(collective_id=0),
    )(pq)


# device time: 35064 ns/iter; 2.2906x vs baseline; 1.0926x over previous
import functools

import jax
import jax.numpy as jnp
from jax import lax
from jax.experimental import pallas as pl
from jax.experimental.pallas import tpu as pltpu

T = 1024
D = 1024
V_PER = 8192
NQ = 4
Q = T // NQ
C = 4
CR = Q // C
H = C // 2


def _ring_to_xz(t):
    tx = t // 2
    tz = (tx + t) % 2
    return tx, tz


def kernel(ids, E):
    my_x = lax.axis_index("x")
    my_y = lax.axis_index("y")
    my_z = lax.axis_index("z")
    r = 2 * my_x + (my_x + my_z) % 2

    my_ids = lax.dynamic_slice(ids, (r * Q,), (Q,))
    local = my_ids - my_y * V_PER
    mask = (local >= 0) & (local < V_PER)
    safe = jnp.where(mask, local, 0).astype(jnp.int32)
    maskf = mask.astype(jnp.float32)[:, None]

    def body(safe_ref, maskf_ref, e_ref, o_ref, gbuf, rbuf, qbuf,
             gsem, ysend, yrecv, s1rs, s1rr, s1ls, s1lr, s2s, s2r):
        my_x = lax.axis_index("x")
        my_y = lax.axis_index("y")
        my_z = lax.axis_index("z")

        r = 2 * my_x + (my_x + my_z) % 2
        rt = (r + 1) % NQ
        lt = (r + 3) % NQ

        rx, rz = _ring_to_xz(rt)
        lx, lz = _ring_to_xz(lt)
        y_peer = (my_x, 1 - my_y, my_z)
        right = (rx, my_y, rz)
        left = (lx, my_y, lz)

        for c in range(C):
            for i in range(c * CR, (c + 1) * CR):
                pltpu.make_async_copy(
                    e_ref.at[pl.ds(safe_ref[i], 1)],
                    gbuf.at[pl.ds(i, 1)],
                    gsem.at[c],
                ).start()

        barrier = pltpu.get_barrier_semaphore()
        for dev in (y_peer, right, left):
            pl.semaphore_signal(
                barrier, inc=1, device_id=dev,
                device_id_type=pl.DeviceIdType.MESH,
            )
        pl.semaphore_wait(barrier, 3)

        y_rdmas = []
        for c in range(C):
            sl = pl.ds(c * CR, CR)
            pltpu.make_async_copy(
                e_ref.at[pl.ds(0, CR)], gbuf.at[sl], gsem.at[c]
            ).wait()
            gbuf[c * CR:(c + 1) * CR, :] = (
                gbuf[c * CR:(c + 1) * CR, :]
                * maskf_ref[c * CR:(c + 1) * CR, :]
            )
            rd = pltpu.make_async_remote_copy(
                src_ref=gbuf.at[sl],
                dst_ref=rbuf.at[sl],
                send_sem=ysend.at[c],
                recv_sem=yrecv.at[c],
                device_id=y_peer,
                device_id_type=pl.DeviceIdType.MESH,
            )
            rd.start()
            y_rdmas.append(rd)

        s1r_rdmas = []
        s1l_rdmas = []
        for c in range(C):
            sl = pl.ds(c * CR, CR)
            y_rdmas[c].wait_recv()
            qbuf[r, c * CR:(c + 1) * CR, :] = (
                gbuf[c * CR:(c + 1) * CR, :] + rbuf[c * CR:(c + 1) * CR, :]
            )
            rdr = pltpu.make_async_remote_copy(
                src_ref=qbuf.at[r, sl],
                dst_ref=qbuf.at[r, sl],
                send_sem=s1rs.at[c],
                recv_sem=s1rr.at[c],
                device_id=right,
                device_id_type=pl.DeviceIdType.MESH,
            )
            rdl = pltpu.make_async_remote_copy(
                src_ref=qbuf.at[r, sl],
                dst_ref=qbuf.at[r, sl],
                send_sem=s1ls.at[c],
                recv_sem=s1lr.at[c],
                device_id=left,
                device_id_type=pl.DeviceIdType.MESH,
            )
            rdr.start()
            rdl.start()
            s1r_rdmas.append(rdr)
            s1l_rdmas.append(rdl)

        s2_rdmas = []
        for c in range(C):
            sl = pl.ds(c * CR, CR)
            if c < H:
                s1r_rdmas[c].wait_recv()
                rd = pltpu.make_async_remote_copy(
                    src_ref=qbuf.at[lt, sl],
                    dst_ref=qbuf.at[lt, sl],
                    send_sem=s2s.at[c],
                    recv_sem=s2r.at[c],
                    device_id=right,
                    device_id_type=pl.DeviceIdType.MESH,
                )
            else:
                s1l_rdmas[c].wait_recv()
                rd = pltpu.make_async_remote_copy(
                    src_ref=qbuf.at[rt, sl],
                    dst_ref=qbuf.at[rt, sl],
                    send_sem=s2s.at[c],
                    recv_sem=s2r.at[c],
                    device_id=left,
                    device_id_type=pl.DeviceIdType.MESH,
                )
            rd.start()
            s2_rdmas.append(rd)

        for c in range(C):
            if c < H:
                s1l_rdmas[c].wait_recv()
            else:
                s1r_rdmas[c].wait_recv()
            s2_rdmas[c].wait_recv()
            y_rdmas[c].wait_send()
            s1r_rdmas[c].wait_send()
            s1l_rdmas[c].wait_send()
            s2_rdmas[c].wait_send()

        for q in range(NQ):
            o_ref[q * Q:(q + 1) * Q, :] = qbuf[q, :, :]

        @functools.partial(
            pl.run_scoped, exit_sem=pltpu.SemaphoreType.REGULAR
        )
        def _(exit_sem):
            for dev in (y_peer, right, left):
                pl.semaphore_signal(
                    exit_sem, inc=1, device_id=dev,
                    device_id_type=pl.DeviceIdType.MESH,
                )
            pl.semaphore_wait(exit_sem, 3)

    return pl.pallas_call(
        body,
        out_shape=jax.ShapeDtypeStruct((T, D), jnp.float32),
        in_specs=[
            pl.BlockSpec(memory_space=pltpu.SMEM),
            pl.BlockSpec(memory_space=pltpu.VMEM),
            pl.BlockSpec(memory_space=pl.ANY),
        ],
        out_specs=pl.BlockSpec(memory_space=pltpu.VMEM),
        scratch_shapes=[
            pltpu.VMEM((Q, D), jnp.float32),
            pltpu.VMEM((Q, D), jnp.float32),
            pltpu.VMEM((NQ, Q, D), jnp.float32),
            pltpu.SemaphoreType.DMA((C,)),
            pltpu.SemaphoreType.DMA((C,)),
            pltpu.SemaphoreType.DMA((C,)),
            pltpu.SemaphoreType.DMA((C,)),
            pltpu.SemaphoreType.DMA((C,)),
            pltpu.SemaphoreType.DMA((C,)),
            pltpu.SemaphoreType.DMA((C,)),
            pltpu.SemaphoreType.DMA((C,)),
            pltpu.SemaphoreType.DMA((C,)),
        ],
        compiler_params=pltpu.CompilerParams(collective_id=0),
    )(safe, maskf, E)
